# baseline (speedup 1.0000x reference)
import functools

import jax
import jax.numpy as jnp
from jax.experimental import pallas as pl
from jax.experimental.pallas import tpu as pltpu

B, C, INTER, H, W = 8, 96, 96, 128, 128
K_CLS, K_REG = 9, 36
HW = H * W
ROWS_PER_CHUNK = 64
N_CHUNK = ROWS_PER_CHUNK * W
N_CHUNKS = H // ROWS_PER_CHUNK
PAD_ROWS = 2
HW_PAD = (H + 2 * PAD_ROWS) * W
REG_OFF = 16
M_SMALL = 56
TAPS = [(dy, dx) for dy in range(3) for dx in range(3)]


def _rpn_kernel(x_ref, w_all_ref, b_inter_ref, w_small_ref, b_small_ref,
                cls_ref, reg_ref, xpad_ref):
    x = x_ref[0].reshape(C, HW).astype(jnp.bfloat16)
    xpad_ref[:, 0:PAD_ROWS * W] = jnp.zeros((C, PAD_ROWS * W), jnp.bfloat16)
    xpad_ref[:, PAD_ROWS * W:PAD_ROWS * W + HW] = x
    xpad_ref[:, PAD_ROWS * W + HW:HW_PAD] = jnp.zeros(
        (C, PAD_ROWS * W), jnp.bfloat16)

    lane = jax.lax.broadcasted_iota(jnp.int32, (C, N_CHUNK), 1) & (W - 1)
    left_edge = lane == 0
    right_edge = lane == W - 1

    w_all = w_all_ref[...]
    w_small = w_small_ref[...]
    b_inter = b_inter_ref[...][:, :1]
    b_small = b_small_ref[...][:, :1]

    for c in range(N_CHUNKS):
        pieces = []
        for dy, dx in TAPS:
            start = c * N_CHUNK + (dy + 1) * W + (dx - 1)
            piece = xpad_ref[:, pl.ds(start, N_CHUNK)]
            if dx == 0:
                piece = jnp.where(left_edge, 0.0, piece)
            elif dx == 2:
                piece = jnp.where(right_edge, 0.0, piece)
            pieces.append(piece)
        xcol = jnp.concatenate(pieces, axis=0)

        acc = jnp.dot(w_all, xcol, preferred_element_type=jnp.float32)
        inter = jnp.maximum(acc + b_inter, 0.0).astype(jnp.bfloat16)

        outs = jnp.dot(w_small, inter,
                       preferred_element_type=jnp.float32) + b_small
        cls_c = jax.nn.sigmoid(outs[0:K_CLS])
        reg_c = outs[REG_OFF:REG_OFF + K_REG]

        row0 = c * ROWS_PER_CHUNK
        cls_ref[0, :, pl.ds(row0, ROWS_PER_CHUNK), :] = cls_c.reshape(
            K_CLS, ROWS_PER_CHUNK, W)
        reg_ref[0, :, pl.ds(row0, ROWS_PER_CHUNK), :] = reg_c.reshape(
            K_REG, ROWS_PER_CHUNK, W)


@jax.jit
def kernel(features, W_inter, b_inter, W_cls, b_cls, W_reg, b_reg):
    w_all = jnp.transpose(W_inter, (0, 2, 3, 1)).reshape(
        INTER, 9 * C).astype(jnp.bfloat16)
    w_small = jnp.zeros((M_SMALL, INTER), jnp.float32)
    w_small = w_small.at[0:K_CLS].set(W_cls.reshape(K_CLS, INTER))
    w_small = w_small.at[REG_OFF:REG_OFF + K_REG].set(
        W_reg.reshape(K_REG, INTER))
    w_small = w_small.astype(jnp.bfloat16)
    b_small = jnp.zeros((M_SMALL, W), jnp.float32)
    b_small = b_small.at[0:K_CLS].set(b_cls[:, None])
    b_small = b_small.at[REG_OFF:REG_OFF + K_REG].set(b_reg[:, None])
    b_inter2 = jnp.tile(b_inter[:, None], (1, W))

    cls, reg = pl.pallas_call(
        _rpn_kernel,
        grid=(B,),
        in_specs=[
            pl.BlockSpec((1, C, H, W), lambda b: (b, 0, 0, 0)),
            pl.BlockSpec((INTER, 9 * C), lambda b: (0, 0)),
            pl.BlockSpec((INTER, W), lambda b: (0, 0)),
            pl.BlockSpec((M_SMALL, INTER), lambda b: (0, 0)),
            pl.BlockSpec((M_SMALL, W), lambda b: (0, 0)),
        ],
        out_specs=[
            pl.BlockSpec((1, K_CLS, H, W), lambda b: (b, 0, 0, 0)),
            pl.BlockSpec((1, K_REG, H, W), lambda b: (b, 0, 0, 0)),
        ],
        out_shape=[
            jax.ShapeDtypeStruct((B, K_CLS, H, W), jnp.float32),
            jax.ShapeDtypeStruct((B, K_REG, H, W), jnp.float32),
        ],
        scratch_shapes=[pltpu.VMEM((C, HW_PAD), jnp.bfloat16)],
        compiler_params=pltpu.CompilerParams(
            dimension_semantics=("parallel",)),
    )(features, w_all, b_inter2, w_small, b_small)
    return (cls, reg)


# speedup vs baseline: 1.3716x; 1.3716x over previous
import functools

import jax
import jax.numpy as jnp
from jax.experimental import pallas as pl
from jax.experimental.pallas import tpu as pltpu

B, C, INTER, H, W = 8, 96, 96, 128, 128
K_CLS, K_REG = 9, 36
HW = H * W
ROWS_PER_CHUNK = 32
N_CHUNK = ROWS_PER_CHUNK * W
N_CHUNKS = H // ROWS_PER_CHUNK
PAD_ROWS = 2
HW_PAD = (H + 2 * PAD_ROWS) * W
REG_OFF = 16
M_SMALL = 56
TAPS = [(dy, dx) for dy in range(3) for dx in range(3)]


def _rpn_kernel(x_ref, w_all_ref, b_inter_ref, w_small_ref, b_small_ref,
                cls_ref, reg_ref, xs_ref):
    x = x_ref[0].reshape(C, HW).astype(jnp.bfloat16)
    lane = jax.lax.broadcasted_iota(jnp.int32, (C, HW), 1) & (W - 1)
    zero_head = jnp.zeros((3 * C, 4 * W), jnp.bfloat16)
    xs_ref[:, 0:4 * W] = zero_head
    xs_ref[:, HW:HW_PAD] = jnp.zeros((3 * C, HW_PAD - HW), jnp.bfloat16)
    xs_ref[0 * C:1 * C, pl.ds(PAD_ROWS * W + 1, HW)] = jnp.where(
        lane == W - 1, 0.0, x)
    xs_ref[1 * C:2 * C, pl.ds(PAD_ROWS * W, HW)] = x
    xs_ref[2 * C:3 * C, pl.ds(PAD_ROWS * W - 1, HW)] = jnp.where(
        lane == 0, 0.0, x)

    w_all = w_all_ref[...]
    w_small = w_small_ref[...]
    b_inter = b_inter_ref[...][:, :1]
    b_small = b_small_ref[...][:, :1]

    for c in range(N_CHUNKS):
        base = c * N_CHUNK
        acc = b_inter
        for dy in range(3):
            acc = acc + jnp.dot(
                w_all[dy], xs_ref[:, pl.ds(base + (dy + 1) * W, N_CHUNK)],
                preferred_element_type=jnp.float32)
        inter = jnp.maximum(acc, 0.0).astype(jnp.bfloat16)

        outs = jnp.dot(w_small, inter,
                       preferred_element_type=jnp.float32) + b_small
        cls_c = jax.nn.sigmoid(outs[0:K_CLS])
        reg_c = outs[REG_OFF:REG_OFF + K_REG]

        row0 = c * ROWS_PER_CHUNK
        cls_ref[0, :, pl.ds(row0, ROWS_PER_CHUNK), :] = cls_c.reshape(
            K_CLS, ROWS_PER_CHUNK, W)
        reg_ref[0, :, pl.ds(row0, ROWS_PER_CHUNK), :] = reg_c.reshape(
            K_REG, ROWS_PER_CHUNK, W)


@jax.jit
def kernel(features, W_inter, b_inter, W_cls, b_cls, W_reg, b_reg):
    w_all = jnp.transpose(W_inter, (2, 0, 3, 1)).reshape(
        3, INTER, 3 * C).astype(jnp.bfloat16)
    w_small = jnp.zeros((M_SMALL, INTER), jnp.float32)
    w_small = w_small.at[0:K_CLS].set(W_cls.reshape(K_CLS, INTER))
    w_small = w_small.at[REG_OFF:REG_OFF + K_REG].set(
        W_reg.reshape(K_REG, INTER))
    w_small = w_small.astype(jnp.bfloat16)
    b_small = jnp.zeros((M_SMALL, W), jnp.float32)
    b_small = b_small.at[0:K_CLS].set(b_cls[:, None])
    b_small = b_small.at[REG_OFF:REG_OFF + K_REG].set(b_reg[:, None])
    b_inter2 = jnp.tile(b_inter[:, None], (1, W))

    cls, reg = pl.pallas_call(
        _rpn_kernel,
        grid=(B,),
        in_specs=[
            pl.BlockSpec((1, C, H, W), lambda b: (b, 0, 0, 0)),
            pl.BlockSpec((3, INTER, 3 * C), lambda b: (0, 0, 0)),
            pl.BlockSpec((INTER, W), lambda b: (0, 0)),
            pl.BlockSpec((M_SMALL, INTER), lambda b: (0, 0)),
            pl.BlockSpec((M_SMALL, W), lambda b: (0, 0)),
        ],
        out_specs=[
            pl.BlockSpec((1, K_CLS, H, W), lambda b: (b, 0, 0, 0)),
            pl.BlockSpec((1, K_REG, H, W), lambda b: (b, 0, 0, 0)),
        ],
        out_shape=[
            jax.ShapeDtypeStruct((B, K_CLS, H, W), jnp.float32),
            jax.ShapeDtypeStruct((B, K_REG, H, W), jnp.float32),
        ],
        scratch_shapes=[pltpu.VMEM((3 * C, HW_PAD), jnp.bfloat16)],
        compiler_params=pltpu.CompilerParams(
            dimension_semantics=("parallel",)),
    )(features, w_all, b_inter2, w_small, b_small)
    return (cls, reg)
